# baseline (speedup 1.0000x reference)
import functools

import jax
import jax.numpy as jnp
from jax import lax
from jax.experimental import pallas as pl
from jax.experimental.pallas import tpu as pltpu
from jax.experimental.pallas import tpu_sc as plsc

_BATCH = 16384
_K = 16
_NC = 2
_NS = 16
_NW = _NC * _NS
_BPW = _BATCH // _NW
_CHUNK = 128
_NCHUNK = _BPW // _CHUNK

_mesh = plsc.VectorSubcoreMesh(core_axis_name="c", subcore_axis_name="s")


@functools.partial(
    pl.kernel,
    mesh=_mesh,
    compiler_params=pltpu.CompilerParams(use_tc_tiling_on_sc=False),
    out_type=jax.ShapeDtypeStruct((_NW, _K), jnp.float32),
    scratch_types=[
        pltpu.VMEM((_NCHUNK, _CHUNK), jnp.int32),
        pltpu.VMEM((_BPW, _K), jnp.float32),
        pltpu.VMEM((_BPW, _K), jnp.float32),
        pltpu.VMEM((_K,), jnp.float32),
        pltpu.SemaphoreType.DMA,
        pltpu.SemaphoreType.DMA,
    ],
)
def _sc_loss(emb_hbm, idx_hbm, table_hbm, out_hbm,
             idx_v, rows_v, emb_v, acc_v, gsem, esem):
    wid = lax.axis_index("s") * _NC + lax.axis_index("c")
    base = wid * _BPW

    pltpu.sync_copy(idx_hbm.at[wid], idx_v)
    emb_cp = pltpu.async_copy(emb_hbm.at[pl.ds(base, _BPW)], emb_v, esem)
    gathers = []
    for j in range(_NCHUNK):
        gathers.append(
            pltpu.async_copy(
                table_hbm.at[idx_v.at[j]],
                rows_v.at[pl.ds(j * _CHUNK, _CHUNK)],
                gsem,
            )
        )
    emb_cp.wait()
    for cp in gathers:
        cp.wait()

    def body(i, acc):
        d = emb_v[i, :] - rows_v[i, :]
        return acc + d * d

    acc = lax.fori_loop(0, _BPW, body, jnp.zeros((_K,), jnp.float32))
    acc_v[...] = acc
    pltpu.sync_copy(acc_v, out_hbm.at[wid])


def kernel(embeddings, labels, table):
    idx = (labels.astype(jnp.int32) - 1).reshape(_NW, _NCHUNK, _CHUNK)
    partials = _sc_loss(embeddings, idx, table)
    return jnp.sum(partials)
